# Initial kernel scaffold; baseline (speedup 1.0000x reference)
import jax
import jax.numpy as jnp
from jax.experimental import pallas as pl


def kernel(to_emb, token_table, pos_table):
    raise NotImplementedError("write your pallas kernel here")


# speedup vs baseline: 3.9415x; 3.9415x over previous
import math

import jax
import jax.numpy as jnp
from jax import lax
from jax.experimental import pallas as pl
from jax.experimental.pallas import tpu as pltpu
from jax.experimental.pallas import tpu_sc as plsc

NC = 2
NS = 16
NW = NC * NS
LANES = 16


def _make_body(batch, seq, emb, half):
    seq_per_w = batch // NW
    scale = math.sqrt(emb)
    nvec = emb // LANES

    def body(to_emb_hbm, table_hbm, pos_hbm, out_hbm, idx_v, rows_v, pos_v, sem):
        wid = lax.axis_index("s") * NC + lax.axis_index("c")
        seq0 = wid * seq_per_w
        pltpu.sync_copy(pos_hbm, pos_v)

        def seq_body(i, carry):
            s = seq0 + i
            pltpu.sync_copy(to_emb_hbm.at[s], idx_v)
            c0 = pltpu.async_copy(table_hbm.at[idx_v.at[0]],
                                  rows_v.at[pl.ds(0, half)], sem)
            c1 = pltpu.async_copy(table_hbm.at[idx_v.at[1]],
                                  rows_v.at[pl.ds(half, half)], sem)
            c0.wait()
            c1.wait()

            def row_body(r, rc):
                for j in range(nvec):
                    sl = pl.ds(j * LANES, LANES)
                    rows_v[r, sl] = rows_v[r, sl] * scale + pos_v[r, sl]
                return rc

            lax.fori_loop(0, seq, row_body, 0)
            pltpu.sync_copy(rows_v, out_hbm.at[s])
            return carry

        lax.fori_loop(0, seq_per_w, seq_body, 0)

    return body


def kernel(to_emb, token_table, pos_table):
    batch, seq = to_emb.shape
    emb = token_table.shape[1]
    half = seq // 2
    to_emb_r = to_emb.reshape(batch, 2, half)
    pos = pos_table[:seq]

    mesh = plsc.VectorSubcoreMesh(core_axis_name="c", subcore_axis_name="s")
    f = pl.kernel(
        _make_body(batch, seq, emb, half),
        mesh=mesh,
        out_type=jax.ShapeDtypeStruct((batch, seq, emb), jnp.float32),
        scratch_types=[
            pltpu.VMEM((2, half), jnp.int32),
            pltpu.VMEM((seq, emb), jnp.float32),
            pltpu.VMEM((seq, emb), jnp.float32),
            pltpu.SemaphoreType.DMA,
        ],
    )
    return f(to_emb_r, token_table, pos)
